# Initial kernel scaffold; baseline (speedup 1.0000x reference)
import functools

import jax
import jax.numpy as jnp
from jax import lax
from jax.experimental import pallas as pl
from jax.experimental.pallas import tpu as pltpu
from jax.experimental.pallas import tpu_sc as plsc

N = 10000
E = 320000
D = 128

NC = 2
NS = 16
L = 16

_MESH = plsc.VectorSubcoreMesh(core_axis_name="c", subcore_axis_name="s")


_DCH = 80
_DPT = E // NS
_DIT = _DPT // _DCH
_ZCH = 400
_NZC = N // _ZCH


@functools.partial(
    pl.kernel,
    out_type=jax.ShapeDtypeStruct((NC, N), jnp.float32),
    mesh=_MESH,
    scratch_types=[
        pltpu.VMEM((_DCH,), jnp.int32),
        pltpu.VMEM((_DCH,), jnp.float32),
        pltpu.VMEM((_ZCH,), jnp.float32),
        pltpu.VMEM_SHARED((N,), jnp.float32),
    ],
)
def _sc_degrees(eidx_hbm, out_hbm, idx_v, ones_v, zbuf_v, hist_sh):
    c = lax.axis_index("c")
    s = lax.axis_index("s")

    def fill_ones(i, _):
        ones_v[pl.ds(i * L, L)] = jnp.ones((L,), jnp.float32)
        return 0

    lax.fori_loop(0, _DCH // L, fill_ones, 0)

    def fill_zero(i, _):
        zbuf_v[pl.ds(i * L, L)] = jnp.zeros((L,), jnp.float32)
        return 0

    lax.fori_loop(0, _ZCH // L, fill_zero, 0)

    def zero_chunk(j, _):
        ch = s + NS * j

        @pl.when(ch < _NZC)
        def _():
            pltpu.sync_copy(zbuf_v, hist_sh.at[pl.ds(ch * _ZCH, _ZCH)])

        return 0

    lax.fori_loop(0, (_NZC + NS - 1) // NS, zero_chunk, 0)
    plsc.subcore_barrier()

    base = c * E + s * _DPT

    def burst(j, _):
        pltpu.sync_copy(eidx_hbm.at[pl.ds(base + j * _DCH, _DCH)], idx_v)
        pltpu.sync_copy(ones_v, hist_sh.at[idx_v], add=True)
        return 0

    lax.fori_loop(0, _DIT, burst, 0)
    plsc.subcore_barrier()

    def write_chunk(j, _):
        ch = s + NS * j

        @pl.when(ch < _NZC)
        def _():
            pltpu.sync_copy(hist_sh.at[pl.ds(ch * _ZCH, _ZCH)],
                            out_hbm.at[c, pl.ds(ch * _ZCH, _ZCH)])

        return 0

    lax.fori_loop(0, (_NZC + NS - 1) // NS, write_chunk, 0)


_ACH = 80
_APT = E // (NC * NS)
_AIT = _APT // _ACH
_RCH = 80
_NRC = N // _RCH


@functools.partial(
    pl.kernel,
    out_type=jax.ShapeDtypeStruct((NC, N, D), jnp.float32),
    mesh=_MESH,
    scratch_types=[
        pltpu.VMEM((_ACH,), jnp.int32),
        pltpu.VMEM((_ACH,), jnp.int32),
        pltpu.VMEM((_ACH, D), jnp.float32),
        pltpu.VMEM((_RCH, D), jnp.float32),
        pltpu.VMEM_SHARED((N, D), jnp.float32),
        pltpu.SemaphoreType.DMA,
    ],
)
def _sc_aggregate(h_hbm, src_hbm, dst_hbm, out_hbm,
                  sidx_v, didx_v, rows_v, zbuf_v, agg_sh, sem):
    c = lax.axis_index("c")
    s = lax.axis_index("s")

    def fill_zero(k, _):
        zbuf_v[k // (D // L), pl.ds((k % (D // L)) * L, L)] = (
            jnp.zeros((L,), jnp.float32))
        return 0

    lax.fori_loop(0, _RCH * (D // L), fill_zero, 0)

    def zero_chunk(j, _):
        ch = s + NS * j

        @pl.when(ch < _NRC)
        def _():
            pltpu.sync_copy(zbuf_v, agg_sh.at[pl.ds(ch * _RCH, _RCH)])

        return 0

    lax.fori_loop(0, (_NRC + NS - 1) // NS, zero_chunk, 0)
    plsc.subcore_barrier()

    base = c * (E // NC) + s * _APT

    def burst(j, _):
        off = base + j * _ACH
        pltpu.sync_copy(src_hbm.at[pl.ds(off, _ACH)], sidx_v)
        pltpu.sync_copy(dst_hbm.at[pl.ds(off, _ACH)], didx_v)
        pltpu.async_copy(h_hbm.at[sidx_v], rows_v, sem).wait()
        pltpu.sync_copy(rows_v, agg_sh.at[didx_v], add=True)
        return 0

    lax.fori_loop(0, _AIT, burst, 0)
    plsc.subcore_barrier()

    def write_chunk(j, _):
        ch = s + NS * j

        @pl.when(ch < _NRC)
        def _():
            pltpu.sync_copy(agg_sh.at[pl.ds(ch * _RCH, _RCH)],
                            out_hbm.at[c, pl.ds(ch * _RCH, _RCH)])

        return 0

    lax.fori_loop(0, (_NRC + NS - 1) // NS, write_chunk, 0)


_BM = 200


def _tc_scale_mm_body(f_ref, d_ref, w_ref, o_ref):
    norm = lax.rsqrt(jnp.maximum(d_ref[...], 1.0))
    h = f_ref[...] * norm
    o_ref[...] = jnp.dot(h, w_ref[...], preferred_element_type=jnp.float32)


def _tc_finish_body(p_ref, d_ref, b_ref, o_ref):
    agg = p_ref[0] + p_ref[1]
    norm = lax.rsqrt(jnp.maximum(d_ref[...], 1.0))
    o_ref[...] = jnp.maximum(agg * norm + b_ref[...], 0.0)


_tc_scale_mm = pl.pallas_call(
    _tc_scale_mm_body,
    grid=(N // _BM,),
    in_specs=[
        pl.BlockSpec((_BM, D), lambda i: (i, 0)),
        pl.BlockSpec((_BM, 1), lambda i: (i, 0)),
        pl.BlockSpec((D, D), lambda i: (0, 0)),
    ],
    out_specs=pl.BlockSpec((_BM, D), lambda i: (i, 0)),
    out_shape=jax.ShapeDtypeStruct((N, D), jnp.float32),
)

_tc_finish = pl.pallas_call(
    _tc_finish_body,
    grid=(N // _BM,),
    in_specs=[
        pl.BlockSpec((NC, _BM, D), lambda i: (0, i, 0)),
        pl.BlockSpec((_BM, 1), lambda i: (i, 0)),
        pl.BlockSpec((1, D), lambda i: (0, 0)),
    ],
    out_specs=pl.BlockSpec((_BM, D), lambda i: (i, 0)),
    out_shape=jax.ShapeDtypeStruct((N, D), jnp.float32),
)


@jax.jit
def kernel(feat, edge_index, W, b):
    eidx = edge_index.astype(jnp.int32).reshape(2 * E)
    src = eidx[:E]
    dst = eidx[E:]
    degs = _sc_degrees(eidx)
    deg_out = degs[0].reshape(N, 1)
    deg_in = degs[1].reshape(N, 1)
    h = _tc_scale_mm(feat, deg_out, W)
    partials = _sc_aggregate(h, src, dst)
    return _tc_finish(partials, deg_in, b.reshape(1, D))


# speedup vs baseline: 4.6475x; 4.6475x over previous
import functools

import jax
import jax.numpy as jnp
from jax import lax
from jax.experimental import pallas as pl
from jax.experimental.pallas import tpu as pltpu
from jax.experimental.pallas import tpu_sc as plsc

N = 10000
E = 320000
D = 128

NC = 2
NS = 16
L = 16

_MESH = plsc.VectorSubcoreMesh(core_axis_name="c", subcore_axis_name="s")


_DCH = 80
_DPT = E // NS
_DIT = _DPT // _DCH
_ZCH = 400
_NZC = N // _ZCH


@functools.partial(
    pl.kernel,
    out_type=jax.ShapeDtypeStruct((NC * N,), jnp.float32),
    mesh=_MESH,
    scratch_types=[
        pltpu.VMEM((_DCH,), jnp.int32),
        pltpu.VMEM((_DCH,), jnp.float32),
        pltpu.VMEM((_ZCH,), jnp.float32),
        pltpu.VMEM_SHARED((N,), jnp.float32),
    ],
)
def _sc_degrees(eidx_hbm, out_hbm, idx_v, ones_v, zbuf_v, hist_sh):
    c = lax.axis_index("c")
    s = lax.axis_index("s")

    def fill_ones(i, _):
        ones_v[pl.ds(i * L, L)] = jnp.ones((L,), jnp.float32)
        return 0

    lax.fori_loop(0, _DCH // L, fill_ones, 0)

    def fill_zero(i, _):
        zbuf_v[pl.ds(i * L, L)] = jnp.zeros((L,), jnp.float32)
        return 0

    lax.fori_loop(0, _ZCH // L, fill_zero, 0)

    def zero_chunk(j, _):
        ch = s + NS * j

        @pl.when(ch < _NZC)
        def _():
            pltpu.sync_copy(zbuf_v, hist_sh.at[pl.ds(ch * _ZCH, _ZCH)])

        return 0

    lax.fori_loop(0, (_NZC + NS - 1) // NS, zero_chunk, 0)
    plsc.subcore_barrier()

    base = c * E + s * _DPT

    def burst(j, _):
        pltpu.sync_copy(eidx_hbm.at[pl.ds(base + j * _DCH, _DCH)], idx_v)
        pltpu.sync_copy(ones_v, hist_sh.at[idx_v], add=True)
        return 0

    lax.fori_loop(0, _DIT, burst, 0)
    plsc.subcore_barrier()

    def write_chunk(j, _):
        ch = s + NS * j

        @pl.when(ch < _NZC)
        def _():
            pltpu.sync_copy(hist_sh.at[pl.ds(ch * _ZCH, _ZCH)], zbuf_v)
            pltpu.sync_copy(zbuf_v,
                            out_hbm.at[pl.ds(c * N + ch * _ZCH, _ZCH)])

        return 0

    lax.fori_loop(0, (_NZC + NS - 1) // NS, write_chunk, 0)


_ACH = 80
_APT = E // (NC * NS)
_AIT = _APT // _ACH
_RCH = 80
_NRC = N // _RCH


@functools.partial(
    pl.kernel,
    out_type=jax.ShapeDtypeStruct((NC, N, D), jnp.float32),
    mesh=_MESH,
    scratch_types=[
        pltpu.VMEM((_ACH,), jnp.int32),
        pltpu.VMEM((_ACH,), jnp.int32),
        pltpu.VMEM((_ACH, D), jnp.float32),
        pltpu.VMEM((_RCH, D), jnp.float32),
        pltpu.VMEM_SHARED((N, D), jnp.float32),
        pltpu.SemaphoreType.DMA,
    ],
)
def _sc_aggregate(h_hbm, src_hbm, dst_hbm, out_hbm,
                  sidx_v, didx_v, rows_v, zbuf_v, agg_sh, sem):
    c = lax.axis_index("c")
    s = lax.axis_index("s")

    def fill_zero(k, _):
        zbuf_v[k // (D // L), pl.ds((k % (D // L)) * L, L)] = (
            jnp.zeros((L,), jnp.float32))
        return 0

    lax.fori_loop(0, _RCH * (D // L), fill_zero, 0)

    def zero_chunk(j, _):
        ch = s + NS * j

        @pl.when(ch < _NRC)
        def _():
            pltpu.sync_copy(zbuf_v, agg_sh.at[pl.ds(ch * _RCH, _RCH)])

        return 0

    lax.fori_loop(0, (_NRC + NS - 1) // NS, zero_chunk, 0)
    plsc.subcore_barrier()

    base = c * (E // NC) + s * _APT

    def burst(j, _):
        off = base + j * _ACH
        pltpu.sync_copy(src_hbm.at[pl.ds(off, _ACH)], sidx_v)
        pltpu.sync_copy(dst_hbm.at[pl.ds(off, _ACH)], didx_v)
        pltpu.async_copy(h_hbm.at[sidx_v], rows_v, sem).wait()
        pltpu.sync_copy(rows_v, agg_sh.at[didx_v], add=True)
        return 0

    lax.fori_loop(0, _AIT, burst, 0)
    plsc.subcore_barrier()

    def write_chunk(j, _):
        ch = s + NS * j

        @pl.when(ch < _NRC)
        def _():
            pltpu.sync_copy(agg_sh.at[pl.ds(ch * _RCH, _RCH)], zbuf_v)
            pltpu.sync_copy(zbuf_v, out_hbm.at[c, pl.ds(ch * _RCH, _RCH)])

        return 0

    lax.fori_loop(0, (_NRC + NS - 1) // NS, write_chunk, 0)


_BM = 200


def _tc_scale_mm_body(f_ref, d_ref, w_ref, o_ref):
    norm = lax.rsqrt(jnp.maximum(d_ref[...], 1.0))
    h = f_ref[...] * norm
    o_ref[...] = jnp.dot(h, w_ref[...], preferred_element_type=jnp.float32)


def _tc_finish_body(p_ref, d_ref, b_ref, o_ref):
    agg = p_ref[0] + p_ref[1]
    norm = lax.rsqrt(jnp.maximum(d_ref[...], 1.0))
    o_ref[...] = jnp.maximum(agg * norm + b_ref[...], 0.0)


_tc_scale_mm = pl.pallas_call(
    _tc_scale_mm_body,
    grid=(N // _BM,),
    in_specs=[
        pl.BlockSpec((_BM, D), lambda i: (i, 0)),
        pl.BlockSpec((_BM, 1), lambda i: (i, 0)),
        pl.BlockSpec((D, D), lambda i: (0, 0)),
    ],
    out_specs=pl.BlockSpec((_BM, D), lambda i: (i, 0)),
    out_shape=jax.ShapeDtypeStruct((N, D), jnp.float32),
)

_tc_finish = pl.pallas_call(
    _tc_finish_body,
    grid=(N // _BM,),
    in_specs=[
        pl.BlockSpec((NC, _BM, D), lambda i: (0, i, 0)),
        pl.BlockSpec((_BM, 1), lambda i: (i, 0)),
        pl.BlockSpec((1, D), lambda i: (0, 0)),
    ],
    out_specs=pl.BlockSpec((_BM, D), lambda i: (i, 0)),
    out_shape=jax.ShapeDtypeStruct((N, D), jnp.float32),
)


@jax.jit
def kernel(feat, edge_index, W, b):
    eidx = edge_index.astype(jnp.int32).reshape(2 * E)
    src = eidx[:E]
    dst = eidx[E:]
    degs = _sc_degrees(eidx)
    deg_out = degs[:N].reshape(N, 1)
    deg_in = degs[N:].reshape(N, 1)
    h = _tc_scale_mm(feat, deg_out, W)
    partials = _sc_aggregate(h, src, dst)
    return _tc_finish(partials, deg_in, b.reshape(1, D))


# speedup vs baseline: 7.9202x; 1.7042x over previous
import functools

import jax
import jax.numpy as jnp
from jax import lax
from jax.experimental import pallas as pl
from jax.experimental.pallas import tpu as pltpu
from jax.experimental.pallas import tpu_sc as plsc

N = 10000
E = 320000
D = 128

NC = 2
NS = 16
L = 16
NW = NC * NS

_MESH = plsc.VectorSubcoreMesh(core_axis_name="c", subcore_axis_name="s")

B = 80
JPT = E // (NW * B)

_ZCH = 400
_NZC = N // _ZCH
_RCH = 80
_NRC = N // _RCH


def _fill1d(ref, n, value):
    def body(i, _):
        ref[pl.ds(i * L, L)] = jnp.full((L,), value, jnp.float32)
        return 0
    lax.fori_loop(0, n // L, body, 0)


@functools.partial(
    pl.kernel,
    out_type=jax.ShapeDtypeStruct((NC * N,), jnp.float32),
    mesh=_MESH,
    scratch_types=[
        pltpu.VMEM((2, B), jnp.int32),
        pltpu.VMEM((B,), jnp.float32),
        pltpu.VMEM((_ZCH,), jnp.float32),
        pltpu.VMEM_SHARED((N,), jnp.float32),
        pltpu.SemaphoreType.DMA((2,)),
    ],
)
def _sc_src_degrees(src_hbm, out_hbm, idx_v, ones_v, zbuf_v, hist_sh, isem):
    c = lax.axis_index("c")
    s = lax.axis_index("s")
    base = (c * NS + s) * JPT * B

    _fill1d(ones_v, B, 1.0)
    _fill1d(zbuf_v, _ZCH, 0.0)

    def zero_chunk(j, _):
        ch = s + NS * j

        @pl.when(ch < _NZC)
        def _():
            pltpu.sync_copy(zbuf_v, hist_sh.at[pl.ds(ch * _ZCH, _ZCH)])

        return 0

    lax.fori_loop(0, (_NZC + NS - 1) // NS, zero_chunk, 0)
    plsc.subcore_barrier()

    def load(j):
        b = j % 2
        return pltpu.make_async_copy(
            src_hbm.at[pl.ds(base + j * B, B)], idx_v.at[b], isem.at[b])

    load(0).start()
    load(1).start()

    def burst(j, _):
        load(j).wait()
        pltpu.sync_copy(ones_v, hist_sh.at[idx_v.at[j % 2]], add=True)

        @pl.when(j + 2 < JPT)
        def _():
            load(j + 2).start()

        return 0

    lax.fori_loop(0, JPT, burst, 0)
    plsc.subcore_barrier()

    def write_chunk(j, _):
        ch = s + NS * j

        @pl.when(ch < _NZC)
        def _():
            pltpu.sync_copy(hist_sh.at[pl.ds(ch * _ZCH, _ZCH)], zbuf_v)
            pltpu.sync_copy(zbuf_v,
                            out_hbm.at[pl.ds(c * N + ch * _ZCH, _ZCH)])

        return 0

    lax.fori_loop(0, (_NZC + NS - 1) // NS, write_chunk, 0)


@functools.partial(
    pl.kernel,
    out_type=(
        jax.ShapeDtypeStruct((NC, N, D), jnp.float32),
        jax.ShapeDtypeStruct((NC * N,), jnp.float32),
    ),
    mesh=_MESH,
    scratch_types=[
        pltpu.VMEM((3, B), jnp.int32),
        pltpu.VMEM((3, B), jnp.int32),
        pltpu.VMEM((3, B, D), jnp.float32),
        pltpu.VMEM((B,), jnp.float32),
        pltpu.VMEM((_ZCH,), jnp.float32),
        pltpu.VMEM_SHARED((N, D), jnp.float32),
        pltpu.VMEM_SHARED((N,), jnp.float32),
        pltpu.SemaphoreType.DMA((3,)),
        pltpu.SemaphoreType.DMA((3,)),
        pltpu.SemaphoreType.DMA((3,)),
    ],
)
def _sc_aggregate(h_hbm, src_hbm, dst_hbm, out_hbm, hout_hbm,
                  sidx_v, didx_v, rows_v, ones_v, zbuf_v,
                  agg_sh, hist_sh, ssem, dsem, gsem):
    c = lax.axis_index("c")
    s = lax.axis_index("s")
    base = (c * NS + s) * JPT * B

    _fill1d(ones_v, B, 1.0)
    _fill1d(zbuf_v, _ZCH, 0.0)

    def fill_zero(k, _):
        rows_v[0, k // (D // L), pl.ds((k % (D // L)) * L, L)] = (
            jnp.zeros((L,), jnp.float32))
        return 0

    lax.fori_loop(0, B * (D // L), fill_zero, 0)

    def zero_chunk(j, _):
        ch = s + NS * j

        @pl.when(ch < _NRC)
        def _():
            pltpu.sync_copy(rows_v.at[0],
                            agg_sh.at[pl.ds(ch * _RCH, _RCH)])

        @pl.when(ch < _NZC)
        def _():
            pltpu.sync_copy(zbuf_v, hist_sh.at[pl.ds(ch * _ZCH, _ZCH)])

        return 0

    lax.fori_loop(0, (_NRC + NS - 1) // NS, zero_chunk, 0)
    plsc.subcore_barrier()

    def loads(j):
        b = j % 3
        return (
            pltpu.make_async_copy(
                src_hbm.at[pl.ds(base + j * B, B)], sidx_v.at[b],
                ssem.at[b]),
            pltpu.make_async_copy(
                dst_hbm.at[pl.ds(base + j * B, B)], didx_v.at[b],
                dsem.at[b]),
        )

    def gather(j):
        b = j % 3
        return pltpu.make_async_copy(
            h_hbm.at[sidx_v.at[b]], rows_v.at[b], gsem.at[b])

    for cp in loads(0):
        cp.start()
    for cp in loads(0):
        cp.wait()
    gather(0).start()
    for cp in loads(1):
        cp.start()

    def burst(j, _):
        b = j % 3
        gather(j).wait()

        @pl.when(j + 1 < JPT)
        def _():
            for cp in loads(j + 1):
                cp.wait()
            gather(j + 1).start()

        pltpu.sync_copy(rows_v.at[b], agg_sh.at[didx_v.at[b]], add=True)
        pltpu.sync_copy(ones_v, hist_sh.at[didx_v.at[b]], add=True)

        @pl.when(j + 2 < JPT)
        def _():
            for cp in loads(j + 2):
                cp.start()

        return 0

    lax.fori_loop(0, JPT, burst, 0)
    plsc.subcore_barrier()

    def write_chunk(j, _):
        ch = s + NS * j

        @pl.when(ch < _NRC)
        def _():
            pltpu.sync_copy(agg_sh.at[pl.ds(ch * _RCH, _RCH)],
                            rows_v.at[0])
            pltpu.sync_copy(rows_v.at[0],
                            out_hbm.at[c, pl.ds(ch * _RCH, _RCH)])

        @pl.when(ch < _NZC)
        def _():
            pltpu.sync_copy(hist_sh.at[pl.ds(ch * _ZCH, _ZCH)], zbuf_v)
            pltpu.sync_copy(zbuf_v,
                            hout_hbm.at[pl.ds(c * N + ch * _ZCH, _ZCH)])

        return 0

    lax.fori_loop(0, (_NRC + NS - 1) // NS, write_chunk, 0)


_BM = 200


def _tc_scale_mm_body(f_ref, d_ref, w_ref, o_ref):
    deg = d_ref[0] + d_ref[1]
    norm = lax.rsqrt(jnp.maximum(deg, 1.0))
    h = f_ref[...] * norm
    o_ref[...] = jnp.dot(h, w_ref[...], preferred_element_type=jnp.float32)


def _tc_finish_body(p_ref, d_ref, b_ref, o_ref):
    agg = p_ref[0] + p_ref[1]
    deg = d_ref[0] + d_ref[1]
    norm = lax.rsqrt(jnp.maximum(deg, 1.0))
    o_ref[...] = jnp.maximum(agg * norm + b_ref[...], 0.0)


_tc_scale_mm = pl.pallas_call(
    _tc_scale_mm_body,
    grid=(N // _BM,),
    in_specs=[
        pl.BlockSpec((_BM, D), lambda i: (i, 0)),
        pl.BlockSpec((NC, _BM, 1), lambda i: (0, i, 0)),
        pl.BlockSpec((D, D), lambda i: (0, 0)),
    ],
    out_specs=pl.BlockSpec((_BM, D), lambda i: (i, 0)),
    out_shape=jax.ShapeDtypeStruct((N, D), jnp.float32),
)

_tc_finish = pl.pallas_call(
    _tc_finish_body,
    grid=(N // _BM,),
    in_specs=[
        pl.BlockSpec((NC, _BM, D), lambda i: (0, i, 0)),
        pl.BlockSpec((NC, _BM, 1), lambda i: (0, i, 0)),
        pl.BlockSpec((1, D), lambda i: (0, 0)),
    ],
    out_specs=pl.BlockSpec((_BM, D), lambda i: (i, 0)),
    out_shape=jax.ShapeDtypeStruct((N, D), jnp.float32),
)


@jax.jit
def kernel(feat, edge_index, W, b):
    eidx = edge_index.astype(jnp.int32)
    src = eidx[0]
    dst = eidx[1]
    degs = _sc_src_degrees(src).reshape(NC, N, 1)
    h = _tc_scale_mm(feat, degs, W)
    partials, hist = _sc_aggregate(h, src, dst)
    return _tc_finish(partials, hist.reshape(NC, N, 1), b.reshape(1, D))


# speedup vs baseline: 8.0715x; 1.0191x over previous
import functools

import jax
import jax.numpy as jnp
from jax import lax
from jax.experimental import pallas as pl
from jax.experimental.pallas import tpu as pltpu
from jax.experimental.pallas import tpu_sc as plsc

N = 10000
E = 320000
D = 128

NC = 2
NS = 16
L = 16
NW = NC * NS

_MESH = plsc.VectorSubcoreMesh(core_axis_name="c", subcore_axis_name="s")

BD = 80
JD = E // (NW * BD)

BA = 80
JA = E // (NW * BA)
NJ = N

_ZCH = 400
_NZC = N // _ZCH
_RCH = 80
_NRC = N // _RCH


def _fill1d(ref, n, value):
    def body(i, _):
        ref[pl.ds(i * L, L)] = jnp.full((L,), value, jnp.float32)
        return 0
    lax.fori_loop(0, n // L, body, 0)


@functools.partial(
    pl.kernel,
    out_type=jax.ShapeDtypeStruct((NC * N,), jnp.float32),
    mesh=_MESH,
    scratch_types=[
        pltpu.VMEM((3, BD), jnp.int32),
        pltpu.VMEM((BD,), jnp.float32),
        pltpu.VMEM((_ZCH,), jnp.float32),
        pltpu.VMEM_SHARED((N,), jnp.float32),
        pltpu.SemaphoreType.DMA((3,)),
        pltpu.SemaphoreType.DMA((3,)),
    ],
)
def _sc_src_degrees(src_hbm, out_hbm, idx_v, ones_v, zbuf_v, hist_sh,
                    lsem, hsem):
    c = lax.axis_index("c")
    s = lax.axis_index("s")
    base = (c * NS + s) * JD * BD

    _fill1d(ones_v, BD, 1.0)
    _fill1d(zbuf_v, _ZCH, 0.0)

    def zero_chunk(j, _):
        ch = s + NS * j

        @pl.when(ch < _NZC)
        def _():
            pltpu.sync_copy(zbuf_v, hist_sh.at[pl.ds(ch * _ZCH, _ZCH)])

        return 0

    lax.fori_loop(0, (_NZC + NS - 1) // NS, zero_chunk, 0)
    plsc.subcore_barrier()

    def load(j):
        b = j % 3
        return pltpu.make_async_copy(
            src_hbm.at[pl.ds(base + j * BD, BD)], idx_v.at[b], lsem.at[b])

    def hscat_wait(j):
        b = j % 3
        pltpu.make_async_copy(ones_v, hist_sh.at[idx_v.at[b]],
                              hsem.at[b]).wait()

    load(0).start()
    load(1).start()

    def burst(j, _):
        b = j % 3
        load(j).wait()

        @pl.when(j >= 1)
        def _():
            hscat_wait(j - 1)

        @pl.when(j + 2 < JD)
        def _():
            load(j + 2).start()

        pltpu.async_copy(ones_v, hist_sh.at[idx_v.at[b]], hsem.at[b],
                         add=True)
        return 0

    lax.fori_loop(0, JD, burst, 0)
    hscat_wait(JD - 1)
    plsc.subcore_barrier()

    def write_chunk(j, _):
        ch = s + NS * j

        @pl.when(ch < _NZC)
        def _():
            pltpu.sync_copy(hist_sh.at[pl.ds(ch * _ZCH, _ZCH)], zbuf_v)
            pltpu.sync_copy(zbuf_v,
                            out_hbm.at[pl.ds(c * N + ch * _ZCH, _ZCH)])

        return 0

    lax.fori_loop(0, (_NZC + NS - 1) // NS, write_chunk, 0)


@functools.partial(
    pl.kernel,
    out_type=(
        jax.ShapeDtypeStruct((NC, N, D), jnp.float32),
        jax.ShapeDtypeStruct((NC * N,), jnp.float32),
    ),
    mesh=_MESH,
    scratch_types=[
        pltpu.VMEM((4, BA), jnp.int32),
        pltpu.VMEM((4, BA), jnp.int32),
        pltpu.VMEM((4, BA, D), jnp.float32),
        pltpu.VMEM((BA,), jnp.float32),
        pltpu.VMEM((_ZCH,), jnp.float32),
        pltpu.VMEM_SHARED((NJ, D), jnp.float32),
        pltpu.VMEM_SHARED((NJ,), jnp.float32),
        pltpu.SemaphoreType.DMA((4,)),
        pltpu.SemaphoreType.DMA((4,)),
        pltpu.SemaphoreType.DMA((4,)),
        pltpu.SemaphoreType.DMA((4,)),
        pltpu.SemaphoreType.DMA((4,)),
    ],
)
def _sc_aggregate(h_hbm, src_hbm, dst_hbm, out_hbm, hout_hbm,
                  sidx_v, didx_v, rows_v, ones_v, zbuf_v,
                  agg_sh, hist_sh, ssem, dsem, gsem, asem, hsem):
    c = lax.axis_index("c")
    s = lax.axis_index("s")
    base = (c * NS + s) * JA * BA

    _fill1d(ones_v, BA, 1.0)
    _fill1d(zbuf_v, _ZCH, 0.0)

    def fill_zero(k, _):
        rows_v[0, k // (D // L), pl.ds((k % (D // L)) * L, L)] = (
            jnp.zeros((L,), jnp.float32))
        return 0

    lax.fori_loop(0, _RCH * (D // L), fill_zero, 0)

    def zero_chunk(j, _):
        ch = s + NS * j

        @pl.when(ch < _NRC)
        def _():
            pltpu.sync_copy(rows_v.at[0, pl.ds(0, _RCH)],
                            agg_sh.at[pl.ds(ch * _RCH, _RCH)])

        @pl.when(ch < _NZC)
        def _():
            pltpu.sync_copy(zbuf_v, hist_sh.at[pl.ds(ch * _ZCH, _ZCH)])

        return 0

    lax.fori_loop(0, (_NRC + NS - 1) // NS, zero_chunk, 0)
    plsc.subcore_barrier()

    def loads(j):
        b = j % 4
        return (
            pltpu.make_async_copy(
                src_hbm.at[pl.ds(base + j * BA, BA)], sidx_v.at[b],
                ssem.at[b]),
            pltpu.make_async_copy(
                dst_hbm.at[pl.ds(base + j * BA, BA)], didx_v.at[b],
                dsem.at[b]),
        )

    def gather(j):
        b = j % 4
        return pltpu.make_async_copy(
            h_hbm.at[sidx_v.at[b]], rows_v.at[b], gsem.at[b])

    def scats_start(j):
        b = j % 4
        pltpu.async_copy(rows_v.at[b], agg_sh.at[didx_v.at[b]],
                         asem.at[b], add=True)
        pltpu.async_copy(ones_v, hist_sh.at[didx_v.at[b]],
                         hsem.at[b], add=True)

    def scats_wait(j):
        b = j % 4
        pltpu.make_async_copy(rows_v.at[b], agg_sh.at[didx_v.at[b]],
                              asem.at[b]).wait()
        pltpu.make_async_copy(ones_v, hist_sh.at[didx_v.at[b]],
                              hsem.at[b]).wait()

    for cp in loads(0):
        cp.start()
    for cp in loads(1):
        cp.start()
    for cp in loads(0):
        cp.wait()
    gather(0).start()

    def burst(j, _):
        @pl.when(j >= 2)
        def _():
            scats_wait(j - 2)

        @pl.when(j + 2 < JA)
        def _():
            for cp in loads(j + 2):
                cp.start()

        gather(j).wait()

        @pl.when(j + 1 < JA)
        def _():
            for cp in loads(j + 1):
                cp.wait()
            gather(j + 1).start()

        scats_start(j)
        return 0

    lax.fori_loop(0, JA, burst, 0)
    scats_wait(JA - 2)
    scats_wait(JA - 1)
    plsc.subcore_barrier()

    def write_chunk(j, _):
        ch = s + NS * j

        @pl.when(ch < _NRC)
        def _():
            pltpu.sync_copy(agg_sh.at[pl.ds(ch * _RCH, _RCH)],
                            rows_v.at[0, pl.ds(0, _RCH)])
            pltpu.sync_copy(rows_v.at[0, pl.ds(0, _RCH)],
                            out_hbm.at[c, pl.ds(ch * _RCH, _RCH)])

        @pl.when(ch < _NZC)
        def _():
            pltpu.sync_copy(hist_sh.at[pl.ds(ch * _ZCH, _ZCH)], zbuf_v)
            pltpu.sync_copy(zbuf_v,
                            hout_hbm.at[pl.ds(c * N + ch * _ZCH, _ZCH)])

        return 0

    lax.fori_loop(0, (_NRC + NS - 1) // NS, write_chunk, 0)


_BM = 200


def _tc_scale_mm_body(f_ref, d_ref, w_ref, o_ref):
    deg = d_ref[0] + d_ref[1]
    norm = lax.rsqrt(jnp.maximum(deg, 1.0))
    h = f_ref[...] * norm
    o_ref[...] = jnp.dot(h, w_ref[...], preferred_element_type=jnp.float32)


def _tc_finish_body(p_ref, d_ref, b_ref, o_ref):
    agg = p_ref[0] + p_ref[1]
    deg = d_ref[0] + d_ref[1]
    norm = lax.rsqrt(jnp.maximum(deg, 1.0))
    o_ref[...] = jnp.maximum(agg * norm + b_ref[...], 0.0)


_tc_scale_mm = pl.pallas_call(
    _tc_scale_mm_body,
    grid=(N // _BM,),
    in_specs=[
        pl.BlockSpec((_BM, D), lambda i: (i, 0)),
        pl.BlockSpec((NC, _BM, 1), lambda i: (0, i, 0)),
        pl.BlockSpec((D, D), lambda i: (0, 0)),
    ],
    out_specs=pl.BlockSpec((_BM, D), lambda i: (i, 0)),
    out_shape=jax.ShapeDtypeStruct((N, D), jnp.float32),
)

_tc_finish = pl.pallas_call(
    _tc_finish_body,
    grid=(N // _BM,),
    in_specs=[
        pl.BlockSpec((NC, _BM, D), lambda i: (0, i, 0)),
        pl.BlockSpec((NC, _BM, 1), lambda i: (0, i, 0)),
        pl.BlockSpec((1, D), lambda i: (0, 0)),
    ],
    out_specs=pl.BlockSpec((_BM, D), lambda i: (i, 0)),
    out_shape=jax.ShapeDtypeStruct((N, D), jnp.float32),
)


@jax.jit
def kernel(feat, edge_index, W, b):
    eidx = edge_index.astype(jnp.int32)
    src = eidx[0]
    dst = eidx[1]
    degs = _sc_src_degrees(src).reshape(NC, N, 1)
    h = _tc_scale_mm(feat, degs, W)
    partials, hist = _sc_aggregate(h, src, dst)
    return _tc_finish(partials, hist.reshape(NC, N, 1), b.reshape(1, D))


# speedup vs baseline: 8.3747x; 1.0376x over previous
import functools

import jax
import jax.numpy as jnp
from jax import lax
from jax.experimental import pallas as pl
from jax.experimental.pallas import tpu as pltpu
from jax.experimental.pallas import tpu_sc as plsc

N = 10000
E = 320000
D = 128

NC = 2
NS = 16
L = 16
NW = NC * NS

_MESH = plsc.VectorSubcoreMesh(core_axis_name="c", subcore_axis_name="s")

BD = 80
JD = E // (NW * BD)

BA = 80
JA = E // (NW * BA)
NJ = N

_ZCH = 400
_NZC = N // _ZCH
_RCH = 80
_NRC = N // _RCH


def _fill1d(ref, n, value):
    def body(i, _):
        ref[pl.ds(i * L, L)] = jnp.full((L,), value, jnp.float32)
        return 0
    lax.fori_loop(0, n // L, body, 0)


@functools.partial(
    pl.kernel,
    out_type=jax.ShapeDtypeStruct((NC * N,), jnp.float32),
    mesh=_MESH,
    scratch_types=[
        pltpu.VMEM((3, BD), jnp.int32),
        pltpu.VMEM((BD,), jnp.float32),
        pltpu.VMEM((_ZCH,), jnp.float32),
        pltpu.VMEM_SHARED((N,), jnp.float32),
        pltpu.SemaphoreType.DMA((3,)),
        pltpu.SemaphoreType.DMA((3,)),
    ],
)
def _sc_src_degrees(src_hbm, out_hbm, idx_v, ones_v, zbuf_v, hist_sh,
                    lsem, hsem):
    c = lax.axis_index("c")
    s = lax.axis_index("s")
    base = (c * NS + s) * JD * BD

    _fill1d(ones_v, BD, 1.0)
    _fill1d(zbuf_v, _ZCH, 0.0)

    def zero_chunk(j, _):
        ch = s + NS * j

        @pl.when(ch < _NZC)
        def _():
            pltpu.sync_copy(zbuf_v, hist_sh.at[pl.ds(ch * _ZCH, _ZCH)])

        return 0

    lax.fori_loop(0, (_NZC + NS - 1) // NS, zero_chunk, 0)
    plsc.subcore_barrier()

    def load(j):
        b = j % 3
        return pltpu.make_async_copy(
            src_hbm.at[pl.ds(base + j * BD, BD)], idx_v.at[b], lsem.at[b])

    def hscat_wait(j):
        b = j % 3
        pltpu.make_async_copy(ones_v, hist_sh.at[idx_v.at[b]],
                              hsem.at[b]).wait()

    load(0).start()
    load(1).start()

    def burst(j, _):
        b = j % 3
        load(j).wait()

        @pl.when(j >= 1)
        def _():
            hscat_wait(j - 1)

        @pl.when(j + 2 < JD)
        def _():
            load(j + 2).start()

        pltpu.async_copy(ones_v, hist_sh.at[idx_v.at[b]], hsem.at[b],
                         add=True)
        return 0

    lax.fori_loop(0, JD, burst, 0)
    hscat_wait(JD - 1)
    plsc.subcore_barrier()

    def write_chunk(j, _):
        ch = s + NS * j

        @pl.when(ch < _NZC)
        def _():
            pltpu.sync_copy(hist_sh.at[pl.ds(ch * _ZCH, _ZCH)], zbuf_v)
            pltpu.sync_copy(zbuf_v,
                            out_hbm.at[pl.ds(c * N + ch * _ZCH, _ZCH)])

        return 0

    lax.fori_loop(0, (_NZC + NS - 1) // NS, write_chunk, 0)


@functools.partial(
    pl.kernel,
    out_type=(
        jax.ShapeDtypeStruct((NC, N, D), jnp.float32),
        jax.ShapeDtypeStruct((NC * N,), jnp.float32),
    ),
    mesh=_MESH,
    scratch_types=[
        pltpu.VMEM((4, BA), jnp.int32),
        pltpu.VMEM((4, BA), jnp.int32),
        pltpu.VMEM((4, BA, D), jnp.float32),
        pltpu.VMEM((BA,), jnp.float32),
        pltpu.VMEM((_ZCH,), jnp.float32),
        pltpu.VMEM_SHARED((NJ, D), jnp.float32),
        pltpu.VMEM_SHARED((NJ,), jnp.float32),
        pltpu.SemaphoreType.DMA((4,)),
        pltpu.SemaphoreType.DMA((4,)),
        pltpu.SemaphoreType.DMA((4,)),
        pltpu.SemaphoreType.DMA((4,)),
        pltpu.SemaphoreType.DMA((4,)),
        pltpu.SemaphoreType.DMA,
    ],
)
def _sc_aggregate(h_hbm, edge_hbm, out_hbm, hout_hbm,
                  sidx_v, didx_v, rows_v, ones_v, zbuf_v,
                  agg_sh, hist_sh, ssem, dsem, gsem, asem, hsem, xsem):
    c = lax.axis_index("c")
    s = lax.axis_index("s")
    base = (c * NS + s) * JA * BA

    _fill1d(ones_v, BA, 1.0)
    _fill1d(zbuf_v, _ZCH, 0.0)

    def fill_zero(k, _):
        rows_v[0, k // (D // L), pl.ds((k % (D // L)) * L, L)] = (
            jnp.zeros((L,), jnp.float32))
        return 0

    lax.fori_loop(0, _RCH * (D // L), fill_zero, 0)

    def zero_cp(ch):
        return pltpu.make_async_copy(
            rows_v.at[0, pl.ds(0, _RCH)],
            agg_sh.at[pl.ds(ch * _RCH, _RCH)], xsem)

    def zero_chunk(j, _):
        ch = s + NS * j

        @pl.when(ch < _NRC)
        def _():
            zero_cp(ch).start()

        @pl.when(ch < _NZC)
        def _():
            pltpu.sync_copy(zbuf_v, hist_sh.at[pl.ds(ch * _ZCH, _ZCH)])

        return 0

    def zero_drain(j, _):
        ch = s + NS * j

        @pl.when(ch < _NRC)
        def _():
            zero_cp(ch).wait()

        return 0

    lax.fori_loop(0, (_NRC + NS - 1) // NS, zero_chunk, 0)
    lax.fori_loop(0, (_NRC + NS - 1) // NS, zero_drain, 0)
    plsc.subcore_barrier()

    def loads(j):
        b = j % 4
        return (
            pltpu.make_async_copy(
                edge_hbm.at[pl.ds(base + j * BA, BA)], sidx_v.at[b],
                ssem.at[b]),
            pltpu.make_async_copy(
                edge_hbm.at[pl.ds(E + base + j * BA, BA)], didx_v.at[b],
                dsem.at[b]),
        )

    def gather(j):
        b = j % 4
        return pltpu.make_async_copy(
            h_hbm.at[sidx_v.at[b]], rows_v.at[b], gsem.at[b])

    def scats_start(j):
        b = j % 4
        pltpu.async_copy(rows_v.at[b], agg_sh.at[didx_v.at[b]],
                         asem.at[b], add=True)
        pltpu.async_copy(ones_v, hist_sh.at[didx_v.at[b]],
                         hsem.at[b], add=True)

    def scats_wait(j):
        b = j % 4
        pltpu.make_async_copy(rows_v.at[b], agg_sh.at[didx_v.at[b]],
                              asem.at[b]).wait()
        pltpu.make_async_copy(ones_v, hist_sh.at[didx_v.at[b]],
                              hsem.at[b]).wait()

    for cp in loads(0):
        cp.start()
    for cp in loads(1):
        cp.start()
    for cp in loads(0):
        cp.wait()
    gather(0).start()

    def burst(j, _):
        @pl.when(j >= 2)
        def _():
            scats_wait(j - 2)

        @pl.when(j + 2 < JA)
        def _():
            for cp in loads(j + 2):
                cp.start()

        gather(j).wait()

        @pl.when(j + 1 < JA)
        def _():
            for cp in loads(j + 1):
                cp.wait()
            gather(j + 1).start()

        scats_start(j)
        return 0

    lax.fori_loop(0, JA, burst, 0)
    scats_wait(JA - 2)
    scats_wait(JA - 1)
    plsc.subcore_barrier()

    def write_cp(ch):
        return pltpu.make_async_copy(
            agg_sh.at[pl.ds(ch * _RCH, _RCH)],
            out_hbm.at[c, pl.ds(ch * _RCH, _RCH)], xsem)

    def write_chunk(j, _):
        ch = s + NS * j

        @pl.when(ch < _NRC)
        def _():
            write_cp(ch).start()

        @pl.when(ch < _NZC)
        def _():
            pltpu.sync_copy(hist_sh.at[pl.ds(ch * _ZCH, _ZCH)], zbuf_v)
            pltpu.sync_copy(zbuf_v,
                            hout_hbm.at[pl.ds(c * N + ch * _ZCH, _ZCH)])

        return 0

    def write_drain(j, _):
        ch = s + NS * j

        @pl.when(ch < _NRC)
        def _():
            write_cp(ch).wait()

        return 0

    lax.fori_loop(0, (_NRC + NS - 1) // NS, write_chunk, 0)
    lax.fori_loop(0, (_NRC + NS - 1) // NS, write_drain, 0)


_BM = 200


def _tc_scale_mm_body(f_ref, d_ref, w_ref, o_ref):
    deg = d_ref[0] + d_ref[1]
    norm = lax.rsqrt(jnp.maximum(deg, 1.0))
    h = f_ref[...] * norm
    o_ref[...] = jnp.dot(h, w_ref[...], preferred_element_type=jnp.float32)


def _tc_finish_body(p_ref, d_ref, b_ref, o_ref):
    agg = p_ref[0] + p_ref[1]
    deg = d_ref[0] + d_ref[1]
    norm = lax.rsqrt(jnp.maximum(deg, 1.0))
    o_ref[...] = jnp.maximum(agg * norm + b_ref[...], 0.0)


_tc_scale_mm = pl.pallas_call(
    _tc_scale_mm_body,
    grid=(N // _BM,),
    in_specs=[
        pl.BlockSpec((_BM, D), lambda i: (i, 0)),
        pl.BlockSpec((NC, _BM, 1), lambda i: (0, i, 0)),
        pl.BlockSpec((D, D), lambda i: (0, 0)),
    ],
    out_specs=pl.BlockSpec((_BM, D), lambda i: (i, 0)),
    out_shape=jax.ShapeDtypeStruct((N, D), jnp.float32),
)

_tc_finish = pl.pallas_call(
    _tc_finish_body,
    grid=(N // _BM,),
    in_specs=[
        pl.BlockSpec((NC, _BM, D), lambda i: (0, i, 0)),
        pl.BlockSpec((NC, _BM, 1), lambda i: (0, i, 0)),
        pl.BlockSpec((1, D), lambda i: (0, 0)),
    ],
    out_specs=pl.BlockSpec((_BM, D), lambda i: (i, 0)),
    out_shape=jax.ShapeDtypeStruct((N, D), jnp.float32),
)


@jax.jit
def kernel(feat, edge_index, W, b):
    eflat = edge_index.astype(jnp.int32).reshape(2 * E)
    degs = _sc_src_degrees(eflat).reshape(NC, N, 1)
    h = _tc_scale_mm(feat, degs, W)
    partials, hist = _sc_aggregate(h, eflat)
    return _tc_finish(partials, hist.reshape(NC, N, 1), b.reshape(1, D))
